# baseline (speedup 1.0000x reference)
import math
import functools

import jax
import jax.numpy as jnp
from jax.experimental import pallas as pl
from jax.experimental.pallas import tpu as pltpu


def _round_up(x, m):
    return (x + m - 1) // m * m


def _stream_gather_kernel(ids_ref, pos_ref, off_ref, chunk_ref, out_hbm, acc,
                          sem_out, *, chunk_rows, n_chunks, dh, scale,
                          unroll):
    h = pl.program_id(0)
    c = pl.program_id(1)
    start = off_ref[c]
    end = off_ref[c + 1]
    base_row = c * chunk_rows

    def do_token(i):
        local = ids_ref[i] - base_row
        pos = pos_ref[i]
        acc[pos, 0] = chunk_ref[local, 0] * jnp.float32(scale)

    cnt = end - start
    n_groups = cnt // unroll

    @pl.loop(0, n_groups)
    def _(g):
        i0 = start + g * unroll
        for u in range(unroll):
            do_token(i0 + u)

    @pl.loop(start + n_groups * unroll, end)
    def _(i):
        do_token(i)

    @pl.when(c == n_chunks - 1)
    def _():
        copy = pltpu.make_async_copy(
            acc, out_hbm.at[:, :, pl.ds(h * dh, dh)], sem_out)
        copy.start()
        copy.wait()


def _chunk_group(flat_ids, n_chunks, chunk_rows):
    n = flat_ids.shape[0]
    cidx = flat_ids // chunk_rows
    chunks = jnp.arange(n_chunks, dtype=jnp.int32)
    onehot = (cidx[None, :] == chunks[:, None]).astype(jnp.int32)
    within = jnp.cumsum(onehot, axis=1) * onehot
    counts = jnp.sum(onehot, axis=1)
    off = jnp.concatenate(
        [jnp.zeros(1, jnp.int32), jnp.cumsum(counts)]).astype(jnp.int32)
    rank = jnp.sum(within, axis=0) - 1
    dest = off[cidx] + rank
    order = jnp.zeros(n, jnp.int32).at[dest].set(
        jnp.arange(n, dtype=jnp.int32))
    grouped_ids = flat_ids[order]
    return grouped_ids, order, off


def _streaming_path(flat_ids, table, *, scale):
    V, D = table.shape
    n = flat_ids.shape[0]
    dh = D // 2

    target = max((8 << 20) // (dh * 4), 8)
    chunk_rows = None
    for cand in range(target, min(2 * target, V) + 1, 8):
        if V % cand == 0:
            chunk_rows = cand
            break
    if chunk_rows is None:
        for cand in range(target, 7, -8):
            if V % cand == 0:
                chunk_rows = cand
                break
    if chunk_rows is None:
        chunk_rows = _round_up(V, 8)
    n_chunks = (V + chunk_rows - 1) // chunk_rows

    grouped_ids, order, off = _chunk_group(flat_ids, n_chunks, chunk_rows)
    table3 = table.reshape(V, 1, D)

    chunk_bytes = chunk_rows * dh * 4
    vmem_limit = int(min(n * dh * 4 + 2 * chunk_bytes + (4 << 20), 58 << 20))

    grid_spec = pltpu.PrefetchScalarGridSpec(
        num_scalar_prefetch=3,
        grid=(2, n_chunks),
        in_specs=[
            pl.BlockSpec((chunk_rows, 1, dh), lambda h, c, *_: (c, 0, h)),
        ],
        out_specs=pl.BlockSpec(memory_space=pl.ANY),
        scratch_shapes=[
            pltpu.VMEM((n, 1, dh), table.dtype),
            pltpu.SemaphoreType.DMA,
        ],
    )
    out = pl.pallas_call(
        functools.partial(_stream_gather_kernel, chunk_rows=chunk_rows,
                          n_chunks=n_chunks, dh=dh, scale=scale, unroll=8),
        out_shape=jax.ShapeDtypeStruct((n, 1, D), table.dtype),
        grid_spec=grid_spec,
        compiler_params=pltpu.CompilerParams(
            dimension_semantics=("parallel", "arbitrary"),
            vmem_limit_bytes=vmem_limit,
            disable_bounds_checks=True,
        ),
        name="embedding_stream_gather",
    )(grouped_ids, order, off, table3)
    return out.reshape(n, D)


def _row_gather_kernel(ids_ref, table_hbm, out_ref, sem0, sem1, *, tile,
                       scale):
    V = table_hbm.shape[0]
    base = pl.program_id(0) * tile

    @pl.loop(0, tile // 2)
    def _(tq):
        for u, sem, prio in ((0, sem0, 0), (1, sem1, 1)):
            t = tq * 2 + u
            row = ids_ref[base + t]
            row = jnp.minimum(jnp.maximum(row, 0), V - 1)
            pltpu.async_copy(
                table_hbm.at[pl.ds(row, 1), :],
                out_ref.at[pl.ds(t, 1), :],
                sem,
                priority=prio,
            )

    half = tile // 2
    pltpu.make_async_copy(
        table_hbm.at[pl.ds(0, half), :],
        out_ref.at[pl.ds(0, half), :],
        sem0,
    ).wait()
    pltpu.make_async_copy(
        table_hbm.at[pl.ds(0, half), :],
        out_ref.at[pl.ds(0, half), :],
        sem1,
    ).wait()

    out_ref[...] = out_ref[...] * jnp.float32(scale)


def _row_gather_path(flat_ids, table, *, scale):
    V, D = table.shape
    n = flat_ids.shape[0]
    tile = min(512, n)
    n_pad = _round_up(n, tile)
    if n_pad != n:
        flat_ids = jnp.pad(flat_ids, (0, n_pad - n))
    vmem_limit = int(min(4 * tile * D * 4 + (8 << 20), 56 << 20))
    grid_spec = pltpu.PrefetchScalarGridSpec(
        num_scalar_prefetch=1,
        grid=(n_pad // tile,),
        in_specs=[pl.BlockSpec(memory_space=pl.ANY)],
        out_specs=pl.BlockSpec((tile, D), lambda i, ids: (i, 0)),
        scratch_shapes=[pltpu.SemaphoreType.DMA, pltpu.SemaphoreType.DMA],
    )
    out = pl.pallas_call(
        functools.partial(_row_gather_kernel, tile=tile, scale=scale),
        out_shape=jax.ShapeDtypeStruct((n_pad, D), table.dtype),
        grid_spec=grid_spec,
        compiler_params=pltpu.CompilerParams(
            dimension_semantics=("parallel",),
            vmem_limit_bytes=vmem_limit,
            disable_bounds_checks=True,
        ),
        name="embedding_row_gather",
    )(flat_ids, table)
    return out[:n]


def kernel(x_ids, table):
    B, S = x_ids.shape
    V, D = table.shape
    N = B * S
    scale = math.sqrt(D)

    n_pad = _round_up(N, 16)
    flat_ids = jnp.clip(x_ids.reshape(N).astype(jnp.int32), 0, V - 1)
    if n_pad != N:
        flat_ids = jnp.pad(flat_ids, (0, n_pad - N))

    half_bytes = n_pad * (D // 2) * 4
    use_streaming = (
        D % 256 == 0
        and half_bytes <= (40 << 20)
        and n_pad * D >= V * D // 4)

    if use_streaming:
        out_flat = _streaming_path(flat_ids, table, scale=scale)
    else:
        out_flat = _row_gather_path(flat_ids, table, scale=scale)

    return out_flat[:N].reshape(B, S, D)


# speedup vs baseline: 1.8680x; 1.8680x over previous
import math
import functools

import jax
import jax.numpy as jnp
from jax.experimental import pallas as pl
from jax.experimental.pallas import tpu as pltpu


def _round_up(x, m):
    return (x + m - 1) // m * m


def _gather_scale_kernel(ids_ref, table_hbm, out_ref, sem0, sem1, *, tile,
                         scale):
    V = table_hbm.shape[0]
    base = pl.program_id(0) * tile

    @pl.loop(0, tile // 4)
    def _(tq):
        for u in range(4):
            t = tq * 4 + u
            row = ids_ref[base + t]
            row = jnp.minimum(jnp.maximum(row, 0), V - 1)
            pltpu.async_copy(
                table_hbm.at[pl.ds(row, 1), :],
                out_ref.at[pl.ds(t, 1), :],
                sem0 if u % 2 == 0 else sem1,
                priority=u % 2,
            )

    half = tile // 2
    pltpu.make_async_copy(
        table_hbm.at[pl.ds(0, half), :],
        out_ref.at[pl.ds(0, half), :],
        sem0,
    ).wait()
    pltpu.make_async_copy(
        table_hbm.at[pl.ds(0, half), :],
        out_ref.at[pl.ds(0, half), :],
        sem1,
    ).wait()

    out_ref[...] = out_ref[...] * jnp.float32(scale)


def kernel(x_ids, table):
    B, S = x_ids.shape
    V, D = table.shape
    N = B * S
    scale = math.sqrt(D)

    tile = min(256, _round_up(N, 8))
    if _round_up(N, tile) // tile < 2 and N > 8:
        tile = min(tile, _round_up((N + 1) // 2, 8))
    n_pad = _round_up(N, tile)

    flat_ids = x_ids.reshape(N).astype(jnp.int32)
    if n_pad != N:
        flat_ids = jnp.pad(flat_ids, (0, n_pad - N))

    itemsize = jnp.dtype(table.dtype).itemsize
    vmem_limit = int(min(4 * tile * D * itemsize + (8 << 20), 56 << 20))

    grid_spec = pltpu.PrefetchScalarGridSpec(
        num_scalar_prefetch=1,
        grid=(n_pad // tile,),
        in_specs=[pl.BlockSpec(memory_space=pl.ANY)],
        out_specs=pl.BlockSpec((tile, D), lambda i, ids: (i, 0)),
        scratch_shapes=[pltpu.SemaphoreType.DMA, pltpu.SemaphoreType.DMA],
    )
    out_flat = pl.pallas_call(
        functools.partial(_gather_scale_kernel, tile=tile, scale=scale),
        out_shape=jax.ShapeDtypeStruct((n_pad, D), table.dtype),
        grid_spec=grid_spec,
        compiler_params=pltpu.CompilerParams(
            dimension_semantics=("parallel",),
            vmem_limit_bytes=vmem_limit,
            disable_bounds_checks=True,
        ),
        name="embedding_gather_scale",
    )(flat_ids, table)

    return out_flat[:N].reshape(B, S, D)


# speedup vs baseline: 2.1829x; 1.1686x over previous
import math
import functools

import jax
import jax.numpy as jnp
from jax.experimental import pallas as pl
from jax.experimental.pallas import tpu as pltpu


def _round_up(x, m):
    return (x + m - 1) // m * m


def _gather_scale_kernel(ids_ref, table_hbm, out_ref, sem0, sem1, *, tile,
                         scale):
    V = table_hbm.shape[0]
    base = pl.program_id(0) * tile

    @pl.loop(0, tile // 2)
    def _(tq):
        for u, sem, prio in ((0, sem0, 0), (1, sem1, 1)):
            t = tq * 2 + u
            row = ids_ref[base + t]
            row = jnp.minimum(jnp.maximum(row, 0), V - 1)
            pltpu.async_copy(
                table_hbm.at[pl.ds(row, 1), :],
                out_ref.at[pl.ds(t, 1), :],
                sem,
                priority=prio,
            )

    half = tile // 2
    pltpu.make_async_copy(
        table_hbm.at[pl.ds(0, half), :],
        out_ref.at[pl.ds(0, half), :],
        sem0,
    ).wait()
    pltpu.make_async_copy(
        table_hbm.at[pl.ds(0, half), :],
        out_ref.at[pl.ds(0, half), :],
        sem1,
    ).wait()

    out_ref[...] = out_ref[...] * jnp.float32(scale)


def kernel(x_ids, table):
    B, S = x_ids.shape
    V, D = table.shape
    N = B * S
    scale = math.sqrt(D)

    tile = min(1024, _round_up(N, 8))
    if _round_up(N, tile) // tile < 2 and N > 8:
        tile = min(tile, _round_up((N + 1) // 2, 8))
    n_pad = _round_up(N, tile)

    flat_ids = x_ids.reshape(N).astype(jnp.int32)
    if n_pad != N:
        flat_ids = jnp.pad(flat_ids, (0, n_pad - N))

    itemsize = jnp.dtype(table.dtype).itemsize
    vmem_limit = int(min(4 * tile * D * itemsize + (8 << 20), 56 << 20))

    grid_spec = pltpu.PrefetchScalarGridSpec(
        num_scalar_prefetch=1,
        grid=(n_pad // tile,),
        in_specs=[pl.BlockSpec(memory_space=pl.ANY)],
        out_specs=pl.BlockSpec((tile, D), lambda i, ids: (i, 0)),
        scratch_shapes=[pltpu.SemaphoreType.DMA, pltpu.SemaphoreType.DMA],
    )
    out_flat = pl.pallas_call(
        functools.partial(_gather_scale_kernel, tile=tile, scale=scale),
        out_shape=jax.ShapeDtypeStruct((n_pad, D), table.dtype),
        grid_spec=grid_spec,
        compiler_params=pltpu.CompilerParams(
            dimension_semantics=("parallel",),
            vmem_limit_bytes=vmem_limit,
            disable_bounds_checks=True,
        ),
        name="embedding_gather_scale",
    )(flat_ids, table)

    return out_flat[:N].reshape(B, S, D)


# speedup vs baseline: 2.2624x; 1.0364x over previous
import math
import functools

import jax
import jax.numpy as jnp
from jax.experimental import pallas as pl
from jax.experimental.pallas import tpu as pltpu


def _round_up(x, m):
    return (x + m - 1) // m * m


def _gather_scale_kernel(ids_ref, table_hbm, out_ref, sem0, sem1, *, tile,
                         scale):
    V = table_hbm.shape[0]
    base = pl.program_id(0) * tile

    @pl.loop(0, tile // 2)
    def _(tq):
        for u, sem, prio in ((0, sem0, 0), (1, sem1, 1)):
            t = tq * 2 + u
            row = ids_ref[base + t]
            row = jnp.minimum(jnp.maximum(row, 0), V - 1)
            pltpu.async_copy(
                table_hbm.at[pl.ds(row, 1), :],
                out_ref.at[pl.ds(t, 1), :],
                sem,
                priority=prio,
            )

    half = tile // 2
    pltpu.make_async_copy(
        table_hbm.at[pl.ds(0, half), :],
        out_ref.at[pl.ds(0, half), :],
        sem0,
    ).wait()
    pltpu.make_async_copy(
        table_hbm.at[pl.ds(0, half), :],
        out_ref.at[pl.ds(0, half), :],
        sem1,
    ).wait()

    out_ref[...] = out_ref[...] * jnp.float32(scale)


def kernel(x_ids, table):
    B, S = x_ids.shape
    V, D = table.shape
    N = B * S
    scale = math.sqrt(D)

    tile = min(2048, _round_up(N, 8))
    if _round_up(N, tile) // tile < 2 and N > 8:
        tile = min(tile, _round_up((N + 1) // 2, 8))
    n_pad = _round_up(N, tile)

    flat_ids = x_ids.reshape(N).astype(jnp.int32)
    if n_pad != N:
        flat_ids = jnp.pad(flat_ids, (0, n_pad - N))

    itemsize = jnp.dtype(table.dtype).itemsize
    vmem_limit = int(min(4 * tile * D * itemsize + (8 << 20), 56 << 20))

    grid_spec = pltpu.PrefetchScalarGridSpec(
        num_scalar_prefetch=1,
        grid=(n_pad // tile,),
        in_specs=[pl.BlockSpec(memory_space=pl.ANY)],
        out_specs=pl.BlockSpec((tile, D), lambda i, ids: (i, 0)),
        scratch_shapes=[pltpu.SemaphoreType.DMA, pltpu.SemaphoreType.DMA],
    )
    out_flat = pl.pallas_call(
        functools.partial(_gather_scale_kernel, tile=tile, scale=scale),
        out_shape=jax.ShapeDtypeStruct((n_pad, D), table.dtype),
        grid_spec=grid_spec,
        compiler_params=pltpu.CompilerParams(
            dimension_semantics=("parallel",),
            vmem_limit_bytes=vmem_limit,
            disable_bounds_checks=True,
        ),
        name="embedding_gather_scale",
    )(flat_ids, table)

    return out_flat[:N].reshape(B, S, D)


# speedup vs baseline: 2.2635x; 1.0005x over previous
import math
import functools

import jax
import jax.numpy as jnp
from jax.experimental import pallas as pl
from jax.experimental.pallas import tpu as pltpu


def _round_up(x, m):
    return (x + m - 1) // m * m


def _gather_scale_kernel(ids_ref, table_hbm, out_ref, sem0, sem1, *, tile,
                         scale):
    V = table_hbm.shape[0]
    base = pl.program_id(0) * tile

    @pl.loop(0, tile // 2)
    def _(tq):
        for u, sem, prio in ((0, sem0, 0), (1, sem1, 1)):
            t = tq * 2 + u
            row = ids_ref[base + t]
            row = jnp.minimum(jnp.maximum(row, 0), V - 1)
            pltpu.async_copy(
                table_hbm.at[pl.ds(row, 1), :],
                out_ref.at[pl.ds(t, 1), :],
                sem,
                priority=prio,
            )

    half = tile // 2
    pltpu.make_async_copy(
        table_hbm.at[pl.ds(0, half), :],
        out_ref.at[pl.ds(0, half), :],
        sem0,
    ).wait()
    pltpu.make_async_copy(
        table_hbm.at[pl.ds(0, half), :],
        out_ref.at[pl.ds(0, half), :],
        sem1,
    ).wait()

    out_ref[...] = out_ref[...] * jnp.float32(scale)


def kernel(x_ids, table):
    B, S = x_ids.shape
    V, D = table.shape
    N = B * S
    scale = math.sqrt(D)

    tile = min(4096, _round_up(N, 8))
    if _round_up(N, tile) // tile < 2 and N > 8:
        tile = min(tile, _round_up((N + 1) // 2, 8))
    n_pad = _round_up(N, tile)

    flat_ids = x_ids.reshape(N).astype(jnp.int32)
    if n_pad != N:
        flat_ids = jnp.pad(flat_ids, (0, n_pad - N))

    itemsize = jnp.dtype(table.dtype).itemsize
    vmem_limit = int(min(4 * tile * D * itemsize + (8 << 20), 56 << 20))

    grid_spec = pltpu.PrefetchScalarGridSpec(
        num_scalar_prefetch=1,
        grid=(n_pad // tile,),
        in_specs=[pl.BlockSpec(memory_space=pl.ANY)],
        out_specs=pl.BlockSpec((tile, D), lambda i, ids: (i, 0)),
        scratch_shapes=[pltpu.SemaphoreType.DMA, pltpu.SemaphoreType.DMA],
    )
    out_flat = pl.pallas_call(
        functools.partial(_gather_scale_kernel, tile=tile, scale=scale),
        out_shape=jax.ShapeDtypeStruct((n_pad, D), table.dtype),
        grid_spec=grid_spec,
        compiler_params=pltpu.CompilerParams(
            dimension_semantics=("parallel",),
            vmem_limit_bytes=vmem_limit,
            disable_bounds_checks=True,
        ),
        name="embedding_gather_scale",
    )(flat_ids, table)

    return out_flat[:N].reshape(B, S, D)


# speedup vs baseline: 2.4327x; 1.0748x over previous
import math
import functools

import jax
import jax.numpy as jnp
from jax.experimental import pallas as pl
from jax.experimental.pallas import tpu as pltpu


def _round_up(x, m):
    return (x + m - 1) // m * m


def _gather_scale_kernel(ids_ref, table_hbm, out_ref, sem0, sem1, *, tile,
                         scale):
    V = table_hbm.shape[0]
    base = pl.program_id(0) * tile

    half = tile // 2

    def issue(t, sem, prio):
        row = ids_ref[base + t]
        row = jnp.minimum(jnp.maximum(row, 0), V - 1)
        pltpu.async_copy(
            table_hbm.at[pl.ds(row, 1), :],
            out_ref.at[pl.ds(t, 1), :],
            sem,
            priority=prio,
        )

    @pl.loop(0, half // 2)
    def _(tq):
        for u in range(2):
            issue(tq * 2 + u, sem0, u)

    @pl.loop(half // 2, tile // 2)
    def _(tq):
        for u in range(2):
            issue(tq * 2 + u, sem1, u)

    pltpu.make_async_copy(
        table_hbm.at[pl.ds(0, half), :],
        out_ref.at[pl.ds(0, half), :],
        sem0,
    ).wait()
    out_ref[pl.ds(0, half), :] = (
        out_ref[pl.ds(0, half), :] * jnp.float32(scale))
    pltpu.make_async_copy(
        table_hbm.at[pl.ds(0, half), :],
        out_ref.at[pl.ds(0, half), :],
        sem1,
    ).wait()
    out_ref[pl.ds(half, half), :] = (
        out_ref[pl.ds(half, half), :] * jnp.float32(scale))


def kernel(x_ids, table):
    B, S = x_ids.shape
    V, D = table.shape
    N = B * S
    scale = math.sqrt(D)

    tile = min(2048, _round_up(N, 8))
    if _round_up(N, tile) // tile < 2 and N > 8:
        tile = min(tile, _round_up((N + 1) // 2, 8))
    n_pad = _round_up(N, tile)

    flat_ids = x_ids.reshape(N).astype(jnp.int32)
    if n_pad != N:
        flat_ids = jnp.pad(flat_ids, (0, n_pad - N))

    itemsize = jnp.dtype(table.dtype).itemsize
    vmem_limit = int(min(4 * tile * D * itemsize + (8 << 20), 56 << 20))

    grid_spec = pltpu.PrefetchScalarGridSpec(
        num_scalar_prefetch=1,
        grid=(n_pad // tile,),
        in_specs=[pl.BlockSpec(memory_space=pl.ANY)],
        out_specs=pl.BlockSpec((tile, D), lambda i, ids: (i, 0)),
        scratch_shapes=[pltpu.SemaphoreType.DMA, pltpu.SemaphoreType.DMA],
    )
    out_flat = pl.pallas_call(
        functools.partial(_gather_scale_kernel, tile=tile, scale=scale),
        out_shape=jax.ShapeDtypeStruct((n_pad, D), table.dtype),
        grid_spec=grid_spec,
        compiler_params=pltpu.CompilerParams(
            dimension_semantics=("parallel",),
            vmem_limit_bytes=vmem_limit,
            disable_bounds_checks=True,
        ),
        name="embedding_gather_scale",
    )(flat_ids, table)

    return out_flat[:N].reshape(B, S, D)


# speedup vs baseline: 2.4500x; 1.0071x over previous
import math
import functools

import jax
import jax.numpy as jnp
from jax.experimental import pallas as pl
from jax.experimental.pallas import tpu as pltpu


def _round_up(x, m):
    return (x + m - 1) // m * m


def _gather_scale_kernel(ids_ref, table_hbm, out_ref, sems, *, tile, scale,
                         n_sub):
    V = table_hbm.shape[0]
    base = pl.program_id(0) * tile
    sub = tile // n_sub

    def issue(t, sem, prio):
        row = ids_ref[base + t]
        row = jnp.minimum(jnp.maximum(row, 0), V - 1)
        pltpu.async_copy(
            table_hbm.at[pl.ds(row, 1), :],
            out_ref.at[pl.ds(t, 1), :],
            sem,
            priority=prio,
        )

    for j in range(n_sub):
        @pl.loop(j * sub // 2, (j + 1) * sub // 2)
        def _(tq, j=j):
            for u in range(2):
                issue(tq * 2 + u, sems.at[j], u)

    for j in range(n_sub):
        pltpu.make_async_copy(
            table_hbm.at[pl.ds(0, sub), :],
            out_ref.at[pl.ds(j * sub, sub), :],
            sems.at[j],
        ).wait()
        out_ref[pl.ds(j * sub, sub), :] = (
            out_ref[pl.ds(j * sub, sub), :] * jnp.float32(scale))


def kernel(x_ids, table):
    B, S = x_ids.shape
    V, D = table.shape
    N = B * S
    scale = math.sqrt(D)

    tile = min(2048, _round_up(N, 8))
    if _round_up(N, tile) // tile < 2 and N > 8:
        tile = min(tile, _round_up((N + 1) // 2, 8))
    n_pad = _round_up(N, tile)
    n_sub = 4 if tile % 16 == 0 else (2 if tile % 8 == 0 else 1)

    flat_ids = x_ids.reshape(N).astype(jnp.int32)
    if n_pad != N:
        flat_ids = jnp.pad(flat_ids, (0, n_pad - N))

    itemsize = jnp.dtype(table.dtype).itemsize
    vmem_limit = int(min(4 * tile * D * itemsize + (8 << 20), 56 << 20))

    grid_spec = pltpu.PrefetchScalarGridSpec(
        num_scalar_prefetch=1,
        grid=(n_pad // tile,),
        in_specs=[pl.BlockSpec(memory_space=pl.ANY)],
        out_specs=pl.BlockSpec((tile, D), lambda i, ids: (i, 0)),
        scratch_shapes=[pltpu.SemaphoreType.DMA((n_sub,))],
    )
    out_flat = pl.pallas_call(
        functools.partial(_gather_scale_kernel, tile=tile, scale=scale,
                          n_sub=n_sub),
        out_shape=jax.ShapeDtypeStruct((n_pad, D), table.dtype),
        grid_spec=grid_spec,
        compiler_params=pltpu.CompilerParams(
            dimension_semantics=("parallel",),
            vmem_limit_bytes=vmem_limit,
            disable_bounds_checks=True,
        ),
        name="embedding_gather_scale",
    )(flat_ids, table)

    return out_flat[:N].reshape(B, S, D)


# speedup vs baseline: 2.4520x; 1.0008x over previous
import math
import functools

import jax
import jax.numpy as jnp
from jax.experimental import pallas as pl
from jax.experimental.pallas import tpu as pltpu


def _round_up(x, m):
    return (x + m - 1) // m * m


def _gather_scale_kernel(ids_ref, table_hbm, out_ref, sems, *, tile, scale,
                         n_sub):
    V = table_hbm.shape[0]
    base = pl.program_id(0) * tile
    sub = tile // n_sub

    def issue(t, sem, prio):
        row = ids_ref[base + t]
        row = jnp.minimum(jnp.maximum(row, 0), V - 1)
        pltpu.async_copy(
            table_hbm.at[pl.ds(row, 1), :],
            out_ref.at[pl.ds(t, 1), :],
            sem,
            priority=prio,
        )

    for j in range(n_sub):
        @pl.loop(j * sub // 2, (j + 1) * sub // 2)
        def _(tq, j=j):
            for u in range(2):
                issue(tq * 2 + u, sems.at[j], u)

    for j in range(n_sub):
        pltpu.make_async_copy(
            table_hbm.at[pl.ds(0, sub), :],
            out_ref.at[pl.ds(j * sub, sub), :],
            sems.at[j],
        ).wait()
        out_ref[pl.ds(j * sub, sub), :] = (
            out_ref[pl.ds(j * sub, sub), :] * jnp.float32(scale))


def kernel(x_ids, table):
    B, S = x_ids.shape
    V, D = table.shape
    N = B * S
    scale = math.sqrt(D)

    tile = min(4096, _round_up(N, 8))
    if _round_up(N, tile) // tile < 2 and N > 8:
        tile = min(tile, _round_up((N + 1) // 2, 8))
    n_pad = _round_up(N, tile)
    if tile % 32 == 0:
        n_sub = 8
    elif tile % 16 == 0:
        n_sub = 4
    else:
        n_sub = 2

    flat_ids = x_ids.reshape(N).astype(jnp.int32)
    if n_pad != N:
        flat_ids = jnp.pad(flat_ids, (0, n_pad - N))

    itemsize = jnp.dtype(table.dtype).itemsize
    vmem_limit = int(min(4 * tile * D * itemsize + (8 << 20), 56 << 20))

    grid_spec = pltpu.PrefetchScalarGridSpec(
        num_scalar_prefetch=1,
        grid=(n_pad // tile,),
        in_specs=[pl.BlockSpec(memory_space=pl.ANY)],
        out_specs=pl.BlockSpec((tile, D), lambda i, ids: (i, 0)),
        scratch_shapes=[pltpu.SemaphoreType.DMA((n_sub,))],
    )
    out_flat = pl.pallas_call(
        functools.partial(_gather_scale_kernel, tile=tile, scale=scale,
                          n_sub=n_sub),
        out_shape=jax.ShapeDtypeStruct((n_pad, D), table.dtype),
        grid_spec=grid_spec,
        compiler_params=pltpu.CompilerParams(
            dimension_semantics=("parallel",),
            vmem_limit_bytes=vmem_limit,
            disable_bounds_checks=True,
        ),
        name="embedding_gather_scale",
    )(flat_ids, table)

    return out_flat[:N].reshape(B, S, D)
